# baseline (speedup 1.0000x reference)
import functools

import jax
import jax.numpy as jnp
from jax import lax
from jax.experimental import pallas as pl
from jax.experimental.pallas import tpu as pltpu
from jax.experimental.pallas import tpu_sc as plsc

B, L, D = 4096, 50, 64
V = 100000
NC, NS = 2, 16
NW = NC * NS
DPW = D // NW
LANES = 16
NVEC = B // LANES
UNROLL = 2


@functools.partial(
    pl.kernel,
    out_type=jax.ShapeDtypeStruct((L, D // 8, B // 128, 8, 128), jnp.float32),
    mesh=plsc.VectorSubcoreMesh(core_axis_name="c", subcore_axis_name="s"),
    compiler_params=pltpu.CompilerParams(
        use_tc_tiling_on_sc=False, needs_layout_passes=False
    ),
    scratch_types=(
        [pltpu.VMEM((V,), jnp.float32)]
        + [pltpu.VMEM((B,), jnp.int32) for _ in range(2)]
        + [pltpu.VMEM((B // 128, 128), jnp.float32) for _ in range(2)]
        + [pltpu.VMEM_SHARED((L, B), jnp.int32)]
        + [pltpu.SemaphoreType.DMA for _ in range(5)]
    ),
)
def _sc_lookup(idx_hbm, tab_hbm, out_hbm, row_v, ib0, ib1, ob0, ob1, sidx,
               rsem, is0, is1, os0, os1):
    ibufs, isems = (ib0, ib1), (is0, is1)
    obufs, osems = (ob0, ob1), (os0, os1)
    core = lax.axis_index("c")
    sid = lax.axis_index("s")
    wid = sid * NC + core

    @pl.when(sid == 0)
    def _stage_idx():
        pltpu.sync_copy(idx_hbm, sidx)

    d_first = wid * DPW
    row_h = pltpu.async_copy(tab_hbm.at[d_first], row_v, rsem)
    plsc.subcore_barrier()

    for k in range(DPW):
        d = d_first + k
        dr, dsub = d // 8, d % 8
        if k == 0:
            row_h.wait()
        else:
            pltpu.sync_copy(tab_hbm.at[d], row_v)

        pltpu.async_copy(sidx.at[0], ibufs[0], isems[0])
        pltpu.async_copy(sidx.at[1], ibufs[1], isems[1])

        def pair(p, _):
            for t in range(2):
                l = 2 * p + t
                ib, ob = ibufs[t], obufs[t]
                pltpu.make_async_copy(sidx.at[0], ib, isems[t]).wait()
                @pl.when(p > 0)
                def _drain_out(t=t, ob=ob):
                    pltpu.make_async_copy(
                        ob, out_hbm.at[0, 0, :, 0, :], osems[t]
                    ).wait()

                @plsc.parallel_loop(0, B // 128, step=1, unroll=UNROLL)
                def _gather(c, ib=ib, ob=ob):
                    base = c * 128
                    for u in range(128 // LANES):
                        iv = ib[pl.ds(base + u * LANES, LANES)]
                        ob[c, pl.ds(u * LANES, LANES)] = (
                            plsc.load_gather(row_v, [iv])
                        )
                pltpu.async_copy(
                    ob, out_hbm.at[l, dr, :, dsub, :], osems[t]
                )
                @pl.when(l + 2 < L)
                def _prefetch(l=l, ib=ib, t=t):
                    pltpu.async_copy(sidx.at[l + 2], ib, isems[t])
            return _

        lax.fori_loop(0, L // 2, pair, None)
        for t in range(2):
            pltpu.make_async_copy(
                obufs[t], out_hbm.at[0, 0, :, 0, :], osems[t]
            ).wait()


def kernel(indices, table):
    idx_t = indices.T.astype(jnp.int32)
    tab_t = table.T
    out5 = _sc_lookup(idx_t, tab_t)
    return out5.transpose(2, 4, 0, 1, 3).reshape(B, L, D)


# speedup vs baseline: 1.0538x; 1.0538x over previous
import functools

import jax
import jax.numpy as jnp
from jax import lax
from jax.experimental import pallas as pl
from jax.experimental.pallas import tpu as pltpu
from jax.experimental.pallas import tpu_sc as plsc

B, L, D = 4096, 50, 64
V = 100000
NC, NS = 2, 16
NW = NC * NS
DPW = D // NW
LANES = 16
NVEC = B // LANES
UNROLL = 16


@functools.partial(
    pl.kernel,
    out_type=jax.ShapeDtypeStruct((L, D // 8, B // 128, 8, 128), jnp.float32),
    mesh=plsc.VectorSubcoreMesh(core_axis_name="c", subcore_axis_name="s"),
    compiler_params=pltpu.CompilerParams(
        use_tc_tiling_on_sc=False, needs_layout_passes=False
    ),
    scratch_types=(
        [pltpu.VMEM((V,), jnp.float32)]
        + [pltpu.VMEM((B,), jnp.int32) for _ in range(2)]
        + [pltpu.VMEM((B // 128, 128), jnp.float32) for _ in range(2)]
        + [pltpu.VMEM_SHARED((L, B), jnp.int32)]
        + [pltpu.SemaphoreType.DMA for _ in range(5)]
    ),
)
def _sc_lookup(idx_hbm, tab_hbm, out_hbm, row_v, ib0, ib1, ob0, ob1, sidx,
               rsem, is0, is1, os0, os1):
    ibufs, isems = (ib0, ib1), (is0, is1)
    obufs, osems = (ob0, ob1), (os0, os1)
    core = lax.axis_index("c")
    sid = lax.axis_index("s")
    wid = sid * NC + core

    @pl.when(sid == 0)
    def _stage_idx():
        pltpu.sync_copy(idx_hbm, sidx)

    d_first = wid * DPW
    row_h = pltpu.async_copy(tab_hbm.at[d_first], row_v, rsem)
    plsc.subcore_barrier()

    for k in range(DPW):
        d = d_first + k
        dr, dsub = d // 8, d % 8
        if k == 0:
            row_h.wait()
        else:
            pltpu.sync_copy(tab_hbm.at[d], row_v)

        pltpu.async_copy(sidx.at[0], ibufs[0], isems[0])
        pltpu.async_copy(sidx.at[1], ibufs[1], isems[1])

        def pair(p, _):
            for t in range(2):
                l = 2 * p + t
                ib, ob = ibufs[t], obufs[t]
                pltpu.make_async_copy(sidx.at[0], ib, isems[t]).wait()
                @pl.when(p > 0)
                def _drain_out(t=t, ob=ob):
                    pltpu.make_async_copy(
                        ob, out_hbm.at[0, 0, :, 0, :], osems[t]
                    ).wait()

                @plsc.parallel_loop(0, B, step=LANES, unroll=UNROLL)
                def _gather(off, ib=ib, ob=ob):
                    iv = ib[pl.ds(off, LANES)]
                    ob[off // 128, pl.ds(off % 128, LANES)] = (
                        plsc.load_gather(row_v, [iv])
                    )
                pltpu.async_copy(
                    ob, out_hbm.at[l, dr, :, dsub, :], osems[t]
                )
                @pl.when(l + 2 < L)
                def _prefetch(l=l, ib=ib, t=t):
                    pltpu.async_copy(sidx.at[l + 2], ib, isems[t])
            return _

        lax.fori_loop(0, L // 2, pair, None)
        for t in range(2):
            pltpu.make_async_copy(
                obufs[t], out_hbm.at[0, 0, :, 0, :], osems[t]
            ).wait()


def kernel(indices, table):
    idx_t = indices.T.astype(jnp.int32)
    tab_t = table.T
    out5 = _sc_lookup(idx_t, tab_t)
    return out5.transpose(2, 4, 0, 1, 3).reshape(B, L, D)


# speedup vs baseline: 1.0580x; 1.0040x over previous
import functools

import jax
import jax.numpy as jnp
from jax import lax
from jax.experimental import pallas as pl
from jax.experimental.pallas import tpu as pltpu
from jax.experimental.pallas import tpu_sc as plsc

B, L, D = 4096, 50, 64
V = 100000
NC, NS = 2, 16
NW = NC * NS
DPW = D // NW
LANES = 16
NVEC = B // LANES
UNROLL = 32


@functools.partial(
    pl.kernel,
    out_type=jax.ShapeDtypeStruct((L, D // 8, B // 128, 8, 128), jnp.float32),
    mesh=plsc.VectorSubcoreMesh(core_axis_name="c", subcore_axis_name="s"),
    compiler_params=pltpu.CompilerParams(
        use_tc_tiling_on_sc=False, needs_layout_passes=False
    ),
    scratch_types=(
        [pltpu.VMEM((V,), jnp.float32)]
        + [pltpu.VMEM((B,), jnp.int32) for _ in range(2)]
        + [pltpu.VMEM((B // 128, 128), jnp.float32) for _ in range(2)]
        + [pltpu.VMEM_SHARED((L, B), jnp.int32)]
        + [pltpu.SemaphoreType.DMA for _ in range(5)]
    ),
)
def _sc_lookup(idx_hbm, tab_hbm, out_hbm, row_v, ib0, ib1, ob0, ob1, sidx,
               rsem, is0, is1, os0, os1):
    ibufs, isems = (ib0, ib1), (is0, is1)
    obufs, osems = (ob0, ob1), (os0, os1)
    core = lax.axis_index("c")
    sid = lax.axis_index("s")
    wid = sid * NC + core

    @pl.when(sid == 0)
    def _stage_idx():
        pltpu.sync_copy(idx_hbm, sidx)

    d_first = wid * DPW
    row_h = pltpu.async_copy(tab_hbm.at[d_first], row_v, rsem)
    plsc.subcore_barrier()

    for k in range(DPW):
        d = d_first + k
        dr, dsub = d // 8, d % 8
        if k == 0:
            row_h.wait()
        else:
            pltpu.sync_copy(tab_hbm.at[d], row_v)

        pltpu.async_copy(sidx.at[0], ibufs[0], isems[0])
        pltpu.async_copy(sidx.at[1], ibufs[1], isems[1])

        def pair(p, _):
            for t in range(2):
                l = 2 * p + t
                ib, ob = ibufs[t], obufs[t]
                pltpu.make_async_copy(sidx.at[0], ib, isems[t]).wait()
                @pl.when(p > 0)
                def _drain_out(t=t, ob=ob):
                    pltpu.make_async_copy(
                        ob, out_hbm.at[0, 0, :, 0, :], osems[t]
                    ).wait()

                @plsc.parallel_loop(0, B, step=LANES, unroll=UNROLL)
                def _gather(off, ib=ib, ob=ob):
                    iv = ib[pl.ds(off, LANES)]
                    ob[off // 128, pl.ds(off % 128, LANES)] = (
                        plsc.load_gather(row_v, [iv])
                    )
                pltpu.async_copy(
                    ob, out_hbm.at[l, dr, :, dsub, :], osems[t]
                )
                @pl.when(l + 2 < L)
                def _prefetch(l=l, ib=ib, t=t):
                    pltpu.async_copy(sidx.at[l + 2], ib, isems[t])
            return _

        lax.fori_loop(0, L // 2, pair, None)
        for t in range(2):
            pltpu.make_async_copy(
                obufs[t], out_hbm.at[0, 0, :, 0, :], osems[t]
            ).wait()


def kernel(indices, table):
    idx_t = indices.T.astype(jnp.int32)
    tab_t = table.T
    out5 = _sc_lookup(idx_t, tab_t)
    return out5.transpose(2, 4, 0, 1, 3).reshape(B, L, D)
